# baseline (speedup 1.0000x reference)
import jax
import jax.numpy as jnp
from jax.experimental import pallas as pl

N = 10000
NHEADS = 4
ALPHA = 0.2


def _leaky(x):
    return jnp.where(x >= 0, x, ALPHA * x)


def _gat_layer(h_in, W, a, src, dst):
    h = h_in @ W
    e = jnp.exp(-_leaky(h[src] @ a[: h.shape[1]] + h[dst] @ a[h.shape[1]:]))
    rowsum = jax.ops.segment_sum(e, src, num_segments=N)
    hp = jax.ops.segment_sum(e[:, None] * h[dst], src, num_segments=N)
    hp = hp / (rowsum[:, None] + 1e-16)
    return jax.nn.elu(hp)


def _softmax_body(x_ref, o_ref):
    x = x_ref[...]
    m = jnp.max(x, axis=1, keepdims=True)
    ex = jnp.exp(x - m)
    o_ref[...] = ex / jnp.sum(ex, axis=1, keepdims=True)


def kernel(x, edges, Wl, bl, Wh, ah, Wend, aend):
    src = edges[0]
    dst = edges[1]
    h0 = x @ Wl.T + bl
    heads = jnp.concatenate([_gat_layer(h0, Wh[i], ah[i], src, dst) for i in range(NHEADS)], axis=1)
    out = _gat_layer(heads, Wend, aend, src, dst)
    return pl.pallas_call(
        _softmax_body,
        out_shape=jax.ShapeDtypeStruct(out.shape, out.dtype),
    )(out)


# speedup vs baseline: 4.0987x; 4.0987x over previous
import functools

import jax
import jax.numpy as jnp
from jax import lax
from jax.experimental import pallas as pl
from jax.experimental.pallas import tpu as pltpu
from jax.experimental.pallas import tpu_sc as plsc

N = 10000
E = 320000
D_IN = 128
D_HID = 64
NHEADS = 4
D_OUT = 128
ALPHA = 0.2

N_ACC = 10240
NE_ROWS = N_ACC // 8
DUMMY = N
C = 32
E_PAD = 323584
EPT1 = E_PAD // 16
EPT2 = E_PAD // 32

_F32 = jnp.float32
_I32 = jnp.int32


_R1 = 1024


def _tc_pre_body(x_ref, wlt_ref, bl_ref, wh_ref, a1_ref, t1_ref, s1_ref):
    x = x_ref[...]
    h0 = jnp.dot(x, wlt_ref[...]) + bl_ref[...]
    for c in range(2):
        ha = jnp.dot(h0, wh_ref[2 * c])
        hb = jnp.dot(h0, wh_ref[2 * c + 1])
        tpair = jnp.concatenate([ha, hb], axis=1)
        t1_ref[c] = tpair
        s1_ref[c] = jnp.dot(tpair, a1_ref[c])


def _tc_pre(xp, wlt, blr, wh, a1):
    grid = N_ACC // _R1
    return pl.pallas_call(
        _tc_pre_body,
        grid=(grid,),
        in_specs=[
            pl.BlockSpec((_R1, D_IN), lambda i: (i, 0)),
            pl.BlockSpec((D_IN, D_IN), lambda i: (0, 0)),
            pl.BlockSpec((1, D_IN), lambda i: (0, 0)),
            pl.BlockSpec((NHEADS, D_IN, D_HID), lambda i: (0, 0, 0)),
            pl.BlockSpec((2, D_IN, 4), lambda i: (0, 0, 0)),
        ],
        out_specs=[
            pl.BlockSpec((2, _R1, 128), lambda i: (0, i, 0)),
            pl.BlockSpec((2, _R1, 4), lambda i: (0, i, 0)),
        ],
        out_shape=[
            jax.ShapeDtypeStruct((2, N_ACC, 128), _F32),
            jax.ShapeDtypeStruct((2, N_ACC, 4), _F32),
        ],
    )(xp, wlt, blr, wh, a1)


def _elu(v):
    return jnp.where(v > 0, v, jnp.exp(v) - 1.0)


def _tc_mid_body(at_ref, ae_ref, wend_ref, a2_ref, b_ref, t2_ref, s2_ref):
    acc = jnp.zeros((_R1, 128), _F32)
    for c in range(2):
        hp = at_ref[c]
        denom = jnp.dot(ae_ref[c], b_ref[...]) + 1e-16
        pair = _elu(hp / denom)
        acc = acc + jnp.dot(pair, wend_ref[c])
    t2_ref[...] = acc
    sval = jnp.dot(acc, a2_ref[...])
    s2_ref[0] = sval
    s2_ref[1] = sval


def _tc_mid(acct1, rs1, wend_r, a2, bmat):
    grid = N_ACC // _R1
    return pl.pallas_call(
        _tc_mid_body,
        grid=(grid,),
        in_specs=[
            pl.BlockSpec((2, _R1, 128), lambda i: (0, i, 0)),
            pl.BlockSpec((2, _R1, 2), lambda i: (0, i, 0)),
            pl.BlockSpec((2, 128, 128), lambda i: (0, 0, 0)),
            pl.BlockSpec((128, 4), lambda i: (0, 0)),
            pl.BlockSpec((2, 128), lambda i: (0, 0)),
        ],
        out_specs=[
            pl.BlockSpec((_R1, 128), lambda i: (i, 0)),
            pl.BlockSpec((2, _R1, 4), lambda i: (0, i, 0)),
        ],
        out_shape=[
            jax.ShapeDtypeStruct((N_ACC, 128), _F32),
            jax.ShapeDtypeStruct((2, N_ACC, 4), _F32),
        ],
    )(acct1, rs1, wend_r, a2, bmat)


_R3 = 1000


def _tc_post_body(at_ref, ae_ref, b0_ref, out_ref):
    hp = at_ref[0] + at_ref[1]
    se = ae_ref[0] + ae_ref[1]
    denom = jnp.dot(se, b0_ref[...]) + 1e-16
    o = _elu(hp / denom)
    m = jnp.max(o, axis=1, keepdims=True)
    ex = jnp.exp(o - m)
    out_ref[...] = ex / jnp.sum(ex, axis=1, keepdims=True)


def _tc_post(acct2, rs2, b0):
    grid = N // _R3
    return pl.pallas_call(
        _tc_post_body,
        grid=(grid,),
        in_specs=[
            pl.BlockSpec((2, _R3, 128), lambda i: (0, i, 0)),
            pl.BlockSpec((2, _R3, 2), lambda i: (0, i, 0)),
            pl.BlockSpec((2, 128), lambda i: (0, 0)),
        ],
        out_specs=pl.BlockSpec((_R3, 128), lambda i: (i, 0)),
        out_shape=jax.ShapeDtypeStruct((N, 128), _F32),
    )(acct2, rs2, b0)


def _bcast_lane(v, k):
    idx = jnp.full((16, 1), k, _I32)
    return lax.gather(
        v, idx,
        dimension_numbers=lax.GatherDimensionNumbers(
            offset_dims=(), collapsed_slice_dims=(0,), start_index_map=(0,)),
        slice_sizes=(1,),
        mode=lax.GatherScatterMode.PROMISE_IN_BOUNDS)


def _make_sc_pass(core_split, use_offset, ept):
    nchunks = ept // C
    mesh = plsc.VectorSubcoreMesh(core_axis_name="c", subcore_axis_name="s")
    nrows = N_ACC + NE_ROWS
    rpt = nrows // 16
    nsr = N_ACC // 32
    spt = nsr // 16

    @functools.partial(
        pl.kernel,
        out_type=jax.ShapeDtypeStruct((2, nrows, 128), _F32),
        mesh=mesh,
        compiler_params=pltpu.CompilerParams(needs_layout_passes=False),
        scratch_types=[
            pltpu.VMEM_SHARED((nrows, 128), _F32),
            pltpu.VMEM_SHARED((nsr, 128), _F32),
            pltpu.VMEM((1, C), _I32),
            pltpu.VMEM((1, C), _I32),
            pltpu.VMEM((1, C), _I32),
            pltpu.VMEM((1, C), _I32),
            pltpu.VMEM((1, C), _I32),
            pltpu.VMEM((1, C), _I32),
            pltpu.VMEM((1, C, 128), _F32),
            pltpu.VMEM((1, C, 128), _F32),
            pltpu.VMEM((1, C, 128), _F32),
            pltpu.VMEM((1, C, 128), _F32),
            pltpu.SemaphoreType.DMA,
            pltpu.SemaphoreType.DMA,
            pltpu.SemaphoreType.DMA,
        ],
    )
    def sc_pass(src_r, dst_r, t_r, s_r, acct_o,
                acct_sh, sct_sh, sbuf, dbuf, dbo, sq, dq, eq, ssrow,
                sdrow, drow, erow, semg, semg2, semg3):
        cix = lax.axis_index("c")
        six = lax.axis_index("s")
        rbase = six * rpt

        z16 = jnp.zeros((16,), _F32)
        for r in range(C):
            for j in range(8):
                drow[0, r, pl.ds(j * 16, 16)] = z16
                erow[0, r, pl.ds(j * 16, 16)] = z16
        done = 0
        while done < rpt:
            nrow = min(C, rpt - done)
            pltpu.sync_copy(drow.at[0, pl.ds(0, nrow)],
                            acct_sh.at[pl.ds(rbase + done, nrow)])
            done += nrow
        @pl.when(six < 8)
        def _():
            pltpu.sync_copy(s_r.at[pl.ds(cix * nsr + six * 40, 40)],
                            sct_sh.at[pl.ds(six * 40, 40)])

        plsc.subcore_barrier()

        if core_split:
            ebase = (six * 2 + cix) * ept
        else:
            ebase = six * ept
        off = cix * N_ACC

        def chunk(g, carry):
            base = ebase + g * C
            pltpu.sync_copy(src_r.at[pl.ds(base, C)], sbuf.at[0])
            pltpu.sync_copy(dst_r.at[pl.ds(base, C)], dbuf.at[0])
            for j in range(C // 16):
                sl = pl.ds(j * 16, 16)
                sv = sbuf[0, sl]
                dv = dbuf[0, sl]
                sq[0, sl] = lax.shift_right_logical(sv, 5)
                dq[0, sl] = lax.shift_right_logical(dv, 5)
                eq[0, sl] = N_ACC + lax.shift_right_logical(sv, 3)
                if use_offset:
                    dbo[0, sl] = dv + off
            dg = dbo if use_offset else dbuf
            cp1 = pltpu.async_copy(sct_sh.at[sq.at[0]], ssrow.at[0], semg)
            cp2 = pltpu.async_copy(sct_sh.at[dq.at[0]], sdrow.at[0], semg2)
            cp3 = pltpu.async_copy(t_r.at[dg.at[0]], drow.at[0], semg3)
            cp1.wait()
            cp2.wait()
            cp3.wait()
            z0 = jnp.zeros((16,), _I32)
            for grp in range(C // 16):
                rowv = grp * 16 + lax.iota(_I32, 16)
                svec = sbuf[0, pl.ds(grp * 16, 16)]
                dvec = dbuf[0, pl.ds(grp * 16, 16)]
                sl4 = (svec & 31) * 4
                dl4 = (dvec & 31) * 4
                sa = plsc.load_gather(ssrow, [z0, rowv, sl4])
                sb = plsc.load_gather(ssrow, [z0, rowv, sl4 + 1])
                da = plsc.load_gather(sdrow, [z0, rowv, dl4 + 2])
                db = plsc.load_gather(sdrow, [z0, rowv, dl4 + 3])
                ta = sa + da
                tb = sb + db
                ea = jnp.exp(-jnp.where(ta >= 0, ta, ALPHA * ta))
                eb = jnp.exp(-jnp.where(tb >= 0, tb, ALPHA * tb))
                el = (svec & 7) * 16
                plsc.store_scatter(erow, [z0, rowv, el], ea)
                plsc.store_scatter(erow, [z0, rowv, el + 1], eb)
                for k in range(16):
                    r = grp * 16 + k
                    ka = _bcast_lane(ea, k)
                    kb = _bcast_lane(eb, k)
                    for j in range(4):
                        sl = pl.ds(j * 16, 16)
                        drow[0, r, sl] = drow[0, r, sl] * ka
                    for j in range(4, 8):
                        sl = pl.ds(j * 16, 16)
                        drow[0, r, sl] = drow[0, r, sl] * kb
            pltpu.sync_copy(drow.at[0], acct_sh.at[sbuf.at[0]], add=True)
            pltpu.sync_copy(erow.at[0], acct_sh.at[eq.at[0]], add=True)
            for grp in range(C // 16):
                rowv = grp * 16 + lax.iota(_I32, 16)
                svec = sbuf[0, pl.ds(grp * 16, 16)]
                el = (svec & 7) * 16
                plsc.store_scatter(erow, [z0, rowv, el], z16)
                plsc.store_scatter(erow, [z0, rowv, el + 1], z16)
            return carry

        lax.fori_loop(0, nchunks, chunk, 0)
        plsc.subcore_barrier()
        rsl = pl.ds(rbase, rpt)
        pltpu.sync_copy(acct_sh.at[rsl], acct_o.at[cix, rsl])

    return sc_pass


_sc_l1 = _make_sc_pass(core_split=False, use_offset=True, ept=EPT1)
_sc_l2 = _make_sc_pass(core_split=True, use_offset=False, ept=EPT2)


def _extract_rowsums(acct):
    er = acct[:, N_ACC:, :].reshape(2, NE_ROWS, 8, 16)
    return er[..., :2].reshape(2, N_ACC, 2)


def kernel(x, edges, Wl, bl, Wh, ah, Wend, aend):
    src = edges[0]
    dst = edges[1]
    npad = E_PAD - E
    srcp = jnp.concatenate([src, jnp.full((npad,), DUMMY, _I32)])
    dstp = jnp.concatenate([dst, jnp.full((npad,), DUMMY, _I32)])
    xp = jnp.zeros((N_ACC, D_IN), _F32).at[:N].set(x)
    wlt = Wl.T
    blr = bl.reshape(1, D_IN)

    a1 = jnp.zeros((2, D_IN, 4), _F32)
    for c in range(2):
        a1 = a1.at[c, 0:64, 0].set(ah[2 * c, :64])
        a1 = a1.at[c, 64:128, 1].set(ah[2 * c + 1, :64])
        a1 = a1.at[c, 0:64, 2].set(ah[2 * c, 64:])
        a1 = a1.at[c, 64:128, 3].set(ah[2 * c + 1, 64:])
    a2 = jnp.zeros((D_IN, 4), _F32)
    a2 = a2.at[:, 0].set(aend[:128]).at[:, 1].set(aend[:128])
    a2 = a2.at[:, 2].set(aend[128:]).at[:, 3].set(aend[128:])
    bmat = jnp.zeros((2, 128), _F32)
    bmat = bmat.at[0, :64].set(1.0).at[1, 64:].set(1.0)
    b0 = jnp.zeros((2, 128), _F32).at[0, :].set(1.0)

    t1, s1 = _tc_pre(xp, wlt, blr, Wh, a1)
    s1p = s1.reshape(2 * (N_ACC // 32), 128)
    acct1 = _sc_l1(srcp, dstp, t1.reshape(2 * N_ACC, 128), s1p)
    rs1 = _extract_rowsums(acct1)
    t2, s2 = _tc_mid(acct1[:, :N_ACC, :], rs1, Wend.reshape(2, 128, 128),
                     a2, bmat)
    s2p = s2.reshape(2 * (N_ACC // 32), 128)
    acct2 = _sc_l2(srcp, dstp, t2, s2p)
    rs2 = _extract_rowsums(acct2)
    return _tc_post(acct2[:, :N_ACC, :], rs2, b0)


# speedup vs baseline: 4.1697x; 1.0173x over previous
import functools

import jax
import jax.numpy as jnp
from jax import lax
from jax.experimental import pallas as pl
from jax.experimental.pallas import tpu as pltpu
from jax.experimental.pallas import tpu_sc as plsc

N = 10000
E = 320000
D_IN = 128
D_HID = 64
NHEADS = 4
D_OUT = 128
ALPHA = 0.2

N_ACC = 10240
NE_ROWS = N_ACC // 8
DUMMY = N
C = 32
E_PAD = 323584
EPT1 = E_PAD // 16
EPT2 = E_PAD // 32

_F32 = jnp.float32
_I32 = jnp.int32


_R1 = 1024


def _tc_pre_body(x_ref, wlt_ref, bl_ref, wh_ref, a1_ref, t1_ref, s1_ref):
    x = x_ref[...]
    h0 = jnp.dot(x, wlt_ref[...]) + bl_ref[...]
    for c in range(2):
        ha = jnp.dot(h0, wh_ref[2 * c])
        hb = jnp.dot(h0, wh_ref[2 * c + 1])
        tpair = jnp.concatenate([ha, hb], axis=1)
        t1_ref[c] = tpair
        s1_ref[c] = jnp.dot(tpair, a1_ref[c])


def _tc_pre(xp, wlt, blr, wh, a1):
    grid = N_ACC // _R1
    return pl.pallas_call(
        _tc_pre_body,
        grid=(grid,),
        in_specs=[
            pl.BlockSpec((_R1, D_IN), lambda i: (i, 0)),
            pl.BlockSpec((D_IN, D_IN), lambda i: (0, 0)),
            pl.BlockSpec((1, D_IN), lambda i: (0, 0)),
            pl.BlockSpec((NHEADS, D_IN, D_HID), lambda i: (0, 0, 0)),
            pl.BlockSpec((2, D_IN, 4), lambda i: (0, 0, 0)),
        ],
        out_specs=[
            pl.BlockSpec((2, _R1, 128), lambda i: (0, i, 0)),
            pl.BlockSpec((2, _R1, 4), lambda i: (0, i, 0)),
        ],
        out_shape=[
            jax.ShapeDtypeStruct((2, N_ACC, 128), _F32),
            jax.ShapeDtypeStruct((2, N_ACC, 4), _F32),
        ],
    )(xp, wlt, blr, wh, a1)


def _elu(v):
    return jnp.where(v > 0, v, jnp.exp(v) - 1.0)


def _tc_mid_body(at_ref, ae_ref, wend_ref, a2_ref, b_ref, t2_ref, s2_ref):
    acc = jnp.zeros((_R1, 128), _F32)
    for c in range(2):
        hp = at_ref[c]
        denom = jnp.dot(ae_ref[c], b_ref[...]) + 1e-16
        pair = _elu(hp / denom)
        acc = acc + jnp.dot(pair, wend_ref[c])
    t2_ref[...] = acc
    sval = jnp.dot(acc, a2_ref[...])
    s2_ref[0] = sval
    s2_ref[1] = sval


def _tc_mid(acct1, rs1, wend_r, a2, bmat):
    grid = N_ACC // _R1
    return pl.pallas_call(
        _tc_mid_body,
        grid=(grid,),
        in_specs=[
            pl.BlockSpec((2, _R1, 128), lambda i: (0, i, 0)),
            pl.BlockSpec((2, _R1, 2), lambda i: (0, i, 0)),
            pl.BlockSpec((2, 128, 128), lambda i: (0, 0, 0)),
            pl.BlockSpec((128, 4), lambda i: (0, 0)),
            pl.BlockSpec((2, 128), lambda i: (0, 0)),
        ],
        out_specs=[
            pl.BlockSpec((_R1, 128), lambda i: (i, 0)),
            pl.BlockSpec((2, _R1, 4), lambda i: (0, i, 0)),
        ],
        out_shape=[
            jax.ShapeDtypeStruct((N_ACC, 128), _F32),
            jax.ShapeDtypeStruct((2, N_ACC, 4), _F32),
        ],
    )(acct1, rs1, wend_r, a2, bmat)


_R3 = 1000


def _tc_post_body(at_ref, ae_ref, b0_ref, out_ref):
    hp = at_ref[0] + at_ref[1]
    se = ae_ref[0] + ae_ref[1]
    denom = jnp.dot(se, b0_ref[...]) + 1e-16
    o = _elu(hp / denom)
    m = jnp.max(o, axis=1, keepdims=True)
    ex = jnp.exp(o - m)
    out_ref[...] = ex / jnp.sum(ex, axis=1, keepdims=True)


def _tc_post(acct2, rs2, b0):
    grid = N // _R3
    return pl.pallas_call(
        _tc_post_body,
        grid=(grid,),
        in_specs=[
            pl.BlockSpec((2, _R3, 128), lambda i: (0, i, 0)),
            pl.BlockSpec((2, _R3, 2), lambda i: (0, i, 0)),
            pl.BlockSpec((2, 128), lambda i: (0, 0)),
        ],
        out_specs=pl.BlockSpec((_R3, 128), lambda i: (i, 0)),
        out_shape=jax.ShapeDtypeStruct((N, 128), _F32),
    )(acct2, rs2, b0)


def _bcast_lane(v, k):
    idx = jnp.full((16, 1), k, _I32)
    return lax.gather(
        v, idx,
        dimension_numbers=lax.GatherDimensionNumbers(
            offset_dims=(), collapsed_slice_dims=(0,), start_index_map=(0,)),
        slice_sizes=(1,),
        mode=lax.GatherScatterMode.PROMISE_IN_BOUNDS)


def _make_sc_pass(core_split, use_offset, ept):
    nchunks = ept // C
    mesh = plsc.VectorSubcoreMesh(core_axis_name="c", subcore_axis_name="s")
    nrows = N_ACC + NE_ROWS
    rpt = nrows // 16
    nsr = N_ACC // 32
    spt = nsr // 16

    @functools.partial(
        pl.kernel,
        out_type=jax.ShapeDtypeStruct((2, nrows, 128), _F32),
        mesh=mesh,
        compiler_params=pltpu.CompilerParams(needs_layout_passes=False),
        scratch_types=[
            pltpu.VMEM_SHARED((nrows, 128), _F32),
            pltpu.VMEM_SHARED((nsr, 128), _F32),
            pltpu.VMEM((1, C), _I32),
            pltpu.VMEM((1, C), _I32),
            pltpu.VMEM((1, C), _I32),
            pltpu.VMEM((1, C), _I32),
            pltpu.VMEM((1, C), _I32),
            pltpu.VMEM((1, C), _I32),
            pltpu.VMEM((1, C, 128), _F32),
            pltpu.VMEM((1, C, 128), _F32),
            pltpu.VMEM((1, C, 128), _F32),
            pltpu.VMEM((1, C, 128), _F32),
            pltpu.SemaphoreType.DMA,
            pltpu.SemaphoreType.DMA,
            pltpu.SemaphoreType.DMA,
        ],
    )
    def sc_pass(src_r, dst_r, t_r, s_r, acct_o,
                acct_sh, sct_sh, sbuf, dbuf, dbo, sq, dq, eq, ssrow,
                sdrow, drow, erow, semg, semg2, semg3):
        cix = lax.axis_index("c")
        six = lax.axis_index("s")
        rbase = six * rpt

        z16 = jnp.zeros((16,), _F32)
        for r in range(C):
            for j in range(8):
                drow[0, r, pl.ds(j * 16, 16)] = z16
                erow[0, r, pl.ds(j * 16, 16)] = z16
        done = 0
        while done < rpt:
            nrow = min(C, rpt - done)
            pltpu.sync_copy(drow.at[0, pl.ds(0, nrow)],
                            acct_sh.at[pl.ds(rbase + done, nrow)])
            done += nrow
        @pl.when(six < 8)
        def _():
            pltpu.sync_copy(s_r.at[pl.ds(cix * nsr + six * 40, 40)],
                            sct_sh.at[pl.ds(six * 40, 40)])

        plsc.subcore_barrier()

        if core_split:
            ebase = (six * 2 + cix) * ept
        else:
            ebase = six * ept
        off = cix * N_ACC

        def chunk(g, carry):
            base = ebase + g * C
            pltpu.sync_copy(src_r.at[pl.ds(base, C)], sbuf.at[0])
            pltpu.sync_copy(dst_r.at[pl.ds(base, C)], dbuf.at[0])
            for j in range(C // 16):
                sl = pl.ds(j * 16, 16)
                sv = sbuf[0, sl]
                dv = dbuf[0, sl]
                sq[0, sl] = lax.shift_right_logical(sv, 5)
                dq[0, sl] = lax.shift_right_logical(dv, 5)
                eq[0, sl] = N_ACC + lax.shift_right_logical(sv, 3)
                if use_offset:
                    dbo[0, sl] = dv + off
            dg = dbo if use_offset else dbuf
            cp1 = pltpu.async_copy(sct_sh.at[sq.at[0]], ssrow.at[0], semg)
            cp2 = pltpu.async_copy(sct_sh.at[dq.at[0]], sdrow.at[0], semg2)
            cp3 = pltpu.async_copy(t_r.at[dg.at[0]], drow.at[0], semg3)
            cp1.wait()
            cp2.wait()
            cp3.wait()
            z0 = jnp.zeros((16,), _I32)
            for grp in range(C // 16):
                rowv = grp * 16 + lax.iota(_I32, 16)
                svec = sbuf[0, pl.ds(grp * 16, 16)]
                dvec = dbuf[0, pl.ds(grp * 16, 16)]
                sl4 = (svec & 31) * 4
                dl4 = (dvec & 31) * 4
                sa = plsc.load_gather(ssrow, [z0, rowv, sl4])
                sb = plsc.load_gather(ssrow, [z0, rowv, sl4 + 1])
                da = plsc.load_gather(sdrow, [z0, rowv, dl4 + 2])
                db = plsc.load_gather(sdrow, [z0, rowv, dl4 + 3])
                ta = sa + da
                tb = sb + db
                ea = jnp.exp(-jnp.where(ta >= 0, ta, ALPHA * ta))
                eb = jnp.exp(-jnp.where(tb >= 0, tb, ALPHA * tb))
                el = (svec & 7) * 16
                plsc.store_scatter(erow, [z0, rowv, el], ea)
                plsc.store_scatter(erow, [z0, rowv, el + 1], eb)
                for k in range(16):
                    r = grp * 16 + k
                    ka = _bcast_lane(ea, k)
                    kb = _bcast_lane(eb, k)
                    for j in range(4):
                        sl = pl.ds(j * 16, 16)
                        drow[0, r, sl] = drow[0, r, sl] * ka
                    for j in range(4, 8):
                        sl = pl.ds(j * 16, 16)
                        drow[0, r, sl] = drow[0, r, sl] * kb
            cs1 = pltpu.async_copy(drow.at[0], acct_sh.at[sbuf.at[0]], semg,
                                   add=True)
            cs2 = pltpu.async_copy(erow.at[0], acct_sh.at[eq.at[0]], semg2,
                                   add=True)
            cs1.wait()
            cs2.wait()
            for grp in range(C // 16):
                rowv = grp * 16 + lax.iota(_I32, 16)
                svec = sbuf[0, pl.ds(grp * 16, 16)]
                el = (svec & 7) * 16
                plsc.store_scatter(erow, [z0, rowv, el], z16)
                plsc.store_scatter(erow, [z0, rowv, el + 1], z16)
            return carry

        lax.fori_loop(0, nchunks, chunk, 0)
        plsc.subcore_barrier()
        rsl = pl.ds(rbase, rpt)
        pltpu.sync_copy(acct_sh.at[rsl], acct_o.at[cix, rsl])

    return sc_pass


_sc_l1 = _make_sc_pass(core_split=False, use_offset=True, ept=EPT1)
_sc_l2 = _make_sc_pass(core_split=True, use_offset=False, ept=EPT2)


def _extract_rowsums(acct):
    er = acct[:, N_ACC:, :].reshape(2, NE_ROWS, 8, 16)
    return er[..., :2].reshape(2, N_ACC, 2)


def kernel(x, edges, Wl, bl, Wh, ah, Wend, aend):
    src = edges[0]
    dst = edges[1]
    npad = E_PAD - E
    srcp = jnp.concatenate([src, jnp.full((npad,), DUMMY, _I32)])
    dstp = jnp.concatenate([dst, jnp.full((npad,), DUMMY, _I32)])
    xp = jnp.zeros((N_ACC, D_IN), _F32).at[:N].set(x)
    wlt = Wl.T
    blr = bl.reshape(1, D_IN)

    a1 = jnp.zeros((2, D_IN, 4), _F32)
    for c in range(2):
        a1 = a1.at[c, 0:64, 0].set(ah[2 * c, :64])
        a1 = a1.at[c, 64:128, 1].set(ah[2 * c + 1, :64])
        a1 = a1.at[c, 0:64, 2].set(ah[2 * c, 64:])
        a1 = a1.at[c, 64:128, 3].set(ah[2 * c + 1, 64:])
    a2 = jnp.zeros((D_IN, 4), _F32)
    a2 = a2.at[:, 0].set(aend[:128]).at[:, 1].set(aend[:128])
    a2 = a2.at[:, 2].set(aend[128:]).at[:, 3].set(aend[128:])
    bmat = jnp.zeros((2, 128), _F32)
    bmat = bmat.at[0, :64].set(1.0).at[1, 64:].set(1.0)
    b0 = jnp.zeros((2, 128), _F32).at[0, :].set(1.0)

    t1, s1 = _tc_pre(xp, wlt, blr, Wh, a1)
    s1p = s1.reshape(2 * (N_ACC // 32), 128)
    acct1 = _sc_l1(srcp, dstp, t1.reshape(2 * N_ACC, 128), s1p)
    rs1 = _extract_rowsums(acct1)
    t2, s2 = _tc_mid(acct1[:, :N_ACC, :], rs1, Wend.reshape(2, 128, 128),
                     a2, bmat)
    s2p = s2.reshape(2 * (N_ACC // 32), 128)
    acct2 = _sc_l2(srcp, dstp, t2, s2p)
    rs2 = _extract_rowsums(acct2)
    return _tc_post(acct2[:, :N_ACC, :], rs2, b0)


# speedup vs baseline: 4.9757x; 1.1933x over previous
import functools

import jax
import jax.numpy as jnp
from jax import lax
from jax.experimental import pallas as pl
from jax.experimental.pallas import tpu as pltpu
from jax.experimental.pallas import tpu_sc as plsc

N = 10000
E = 320000
D_IN = 128
D_HID = 64
NHEADS = 4
D_OUT = 128
ALPHA = 0.2

N_ACC = 10240
NE_ROWS = N_ACC // 8
DUMMY = N
C = 32
E_PAD = 323584
EPT1 = E_PAD // 16
EPT2 = E_PAD // 32

_F32 = jnp.float32
_I32 = jnp.int32


_R1 = 1024


def _tc_pre_body(x_ref, wlt_ref, bl_ref, wh_ref, a1_ref, t1_ref, s1_ref):
    x = x_ref[...]
    h0 = jnp.dot(x, wlt_ref[...]) + bl_ref[...]
    for c in range(2):
        ha = jnp.dot(h0, wh_ref[2 * c])
        hb = jnp.dot(h0, wh_ref[2 * c + 1])
        tpair = jnp.concatenate([ha, hb], axis=1)
        t1_ref[c] = tpair
        s1_ref[c] = jnp.dot(tpair, a1_ref[c])


def _tc_pre(xp, wlt, blr, wh, a1):
    grid = N_ACC // _R1
    return pl.pallas_call(
        _tc_pre_body,
        grid=(grid,),
        in_specs=[
            pl.BlockSpec((_R1, D_IN), lambda i: (i, 0)),
            pl.BlockSpec((D_IN, D_IN), lambda i: (0, 0)),
            pl.BlockSpec((1, D_IN), lambda i: (0, 0)),
            pl.BlockSpec((NHEADS, D_IN, D_HID), lambda i: (0, 0, 0)),
            pl.BlockSpec((2, D_IN, 4), lambda i: (0, 0, 0)),
        ],
        out_specs=[
            pl.BlockSpec((2, _R1, 128), lambda i: (0, i, 0)),
            pl.BlockSpec((2, _R1, 4), lambda i: (0, i, 0)),
        ],
        out_shape=[
            jax.ShapeDtypeStruct((2, N_ACC, 128), _F32),
            jax.ShapeDtypeStruct((2, N_ACC, 4), _F32),
        ],
    )(xp, wlt, blr, wh, a1)


def _elu(v):
    return jnp.where(v > 0, v, jnp.exp(v) - 1.0)


def _tc_mid_body(at_ref, ae_ref, wend_ref, a2_ref, b_ref, t2_ref, s2_ref):
    acc = jnp.zeros((_R1, 128), _F32)
    for c in range(2):
        hp = at_ref[c]
        denom = jnp.dot(ae_ref[c], b_ref[...]) + 1e-16
        pair = _elu(hp / denom)
        acc = acc + jnp.dot(pair, wend_ref[c])
    t2_ref[...] = acc
    sval = jnp.dot(acc, a2_ref[...])
    s2_ref[0] = sval
    s2_ref[1] = sval


def _tc_mid(acct1, rs1, wend_r, a2, bmat):
    grid = N_ACC // _R1
    return pl.pallas_call(
        _tc_mid_body,
        grid=(grid,),
        in_specs=[
            pl.BlockSpec((2, _R1, 128), lambda i: (0, i, 0)),
            pl.BlockSpec((2, _R1, 2), lambda i: (0, i, 0)),
            pl.BlockSpec((2, 128, 128), lambda i: (0, 0, 0)),
            pl.BlockSpec((128, 4), lambda i: (0, 0)),
            pl.BlockSpec((2, 128), lambda i: (0, 0)),
        ],
        out_specs=[
            pl.BlockSpec((_R1, 128), lambda i: (i, 0)),
            pl.BlockSpec((2, _R1, 4), lambda i: (0, i, 0)),
        ],
        out_shape=[
            jax.ShapeDtypeStruct((N_ACC, 128), _F32),
            jax.ShapeDtypeStruct((2, N_ACC, 4), _F32),
        ],
    )(acct1, rs1, wend_r, a2, bmat)


_R3 = 1000


def _tc_post_body(at_ref, ae_ref, b0_ref, out_ref):
    hp = at_ref[0] + at_ref[1]
    se = ae_ref[0] + ae_ref[1]
    denom = jnp.dot(se, b0_ref[...]) + 1e-16
    o = _elu(hp / denom)
    m = jnp.max(o, axis=1, keepdims=True)
    ex = jnp.exp(o - m)
    out_ref[...] = ex / jnp.sum(ex, axis=1, keepdims=True)


def _tc_post(acct2, rs2, b0):
    grid = N // _R3
    return pl.pallas_call(
        _tc_post_body,
        grid=(grid,),
        in_specs=[
            pl.BlockSpec((2, _R3, 128), lambda i: (0, i, 0)),
            pl.BlockSpec((2, _R3, 2), lambda i: (0, i, 0)),
            pl.BlockSpec((2, 128), lambda i: (0, 0)),
        ],
        out_specs=pl.BlockSpec((_R3, 128), lambda i: (i, 0)),
        out_shape=jax.ShapeDtypeStruct((N, 128), _F32),
    )(acct2, rs2, b0)


def _bcast_lane(v, k):
    idx = jnp.full((16, 1), k, _I32)
    return lax.gather(
        v, idx,
        dimension_numbers=lax.GatherDimensionNumbers(
            offset_dims=(), collapsed_slice_dims=(0,), start_index_map=(0,)),
        slice_sizes=(1,),
        mode=lax.GatherScatterMode.PROMISE_IN_BOUNDS)


def _make_sc_pass(core_split, use_offset, ept):
    nchunks = ept // C
    mesh = plsc.VectorSubcoreMesh(core_axis_name="c", subcore_axis_name="s")
    nrows = N_ACC + NE_ROWS
    rpt = nrows // 16
    nsr = N_ACC // 32
    spt = nsr // 16

    @functools.partial(
        pl.kernel,
        out_type=jax.ShapeDtypeStruct((2, nrows, 128), _F32),
        mesh=mesh,
        compiler_params=pltpu.CompilerParams(needs_layout_passes=False),
        scratch_types=[
            pltpu.VMEM_SHARED((nrows, 128), _F32),
            pltpu.VMEM_SHARED((nsr, 128), _F32),
            pltpu.VMEM((2, C), _I32),
            pltpu.VMEM((1, C), _I32),
            pltpu.VMEM((1, C), _I32),
            pltpu.VMEM((1, C), _I32),
            pltpu.VMEM((1, C), _I32),
            pltpu.VMEM((2, C), _I32),
            pltpu.VMEM((1, C, 128), _F32),
            pltpu.VMEM((1, C, 128), _F32),
            pltpu.VMEM((2, C, 128), _F32),
            pltpu.VMEM((2, C, 128), _F32),
            pltpu.SemaphoreType.DMA,
            pltpu.SemaphoreType.DMA,
            pltpu.SemaphoreType.DMA,
            pltpu.SemaphoreType.DMA,
            pltpu.SemaphoreType.DMA,
            pltpu.SemaphoreType.DMA,
            pltpu.SemaphoreType.DMA,
        ],
    )
    def sc_pass(src_r, dst_r, t_r, s_r, acct_o,
                acct_sh, sct_sh, sbuf, dbuf, dbo, sq, dq, eq, ssrow,
                sdrow, drow, erow, semg, semg2, semg3,
                sh0, sh1, se0, se1):
        cix = lax.axis_index("c")
        six = lax.axis_index("s")
        rbase = six * rpt

        z16 = jnp.zeros((16,), _F32)
        for r in range(C):
            for j in range(8):
                drow[0, r, pl.ds(j * 16, 16)] = z16
                erow[0, r, pl.ds(j * 16, 16)] = z16
        done = 0
        while done < rpt:
            nrow = min(C, rpt - done)
            pltpu.sync_copy(drow.at[0, pl.ds(0, nrow)],
                            acct_sh.at[pl.ds(rbase + done, nrow)])
            done += nrow
        @pl.when(six < 8)
        def _():
            pltpu.sync_copy(s_r.at[pl.ds(cix * nsr + six * 40, 40)],
                            sct_sh.at[pl.ds(six * 40, 40)])

        plsc.subcore_barrier()

        if core_split:
            ebase = (six * 2 + cix) * ept
        else:
            ebase = six * ept
        off = cix * N_ACC

        hsems = (sh0, sh1)
        esems = (se0, se1)
        z0 = jnp.zeros((16,), _I32)

        def pair(i, carry):
            for b in range(2):
                g = i * 2 + b

                @pl.when(i > 0)
                def _():
                    pltpu.make_async_copy(
                        drow.at[b], acct_sh.at[sbuf.at[b]], hsems[b]).wait()
                    pltpu.make_async_copy(
                        erow.at[b], acct_sh.at[eq.at[b]], esems[b]).wait()
                    for grp in range(C // 16):
                        rowv = grp * 16 + lax.iota(_I32, 16)
                        svec = sbuf[b, pl.ds(grp * 16, 16)]
                        el = (svec & 7) * 16
                        bv = jnp.full((16,), b, _I32)
                        plsc.store_scatter(erow, [bv, rowv, el], z16)
                        plsc.store_scatter(erow, [bv, rowv, el + 1], z16)

                base = ebase + g * C
                pltpu.sync_copy(src_r.at[pl.ds(base, C)], sbuf.at[b])
                pltpu.sync_copy(dst_r.at[pl.ds(base, C)], dbuf.at[0])
                for j in range(C // 16):
                    sl = pl.ds(j * 16, 16)
                    sv = sbuf[b, sl]
                    dv = dbuf[0, sl]
                    sq[0, sl] = lax.shift_right_logical(sv, 5)
                    dq[0, sl] = lax.shift_right_logical(dv, 5)
                    eq[b, sl] = N_ACC + lax.shift_right_logical(sv, 3)
                    if use_offset:
                        dbo[0, sl] = dv + off
                dg = dbo if use_offset else dbuf
                cp1 = pltpu.async_copy(sct_sh.at[sq.at[0]], ssrow.at[0], semg)
                cp2 = pltpu.async_copy(sct_sh.at[dq.at[0]], sdrow.at[0],
                                       semg2)
                cp3 = pltpu.async_copy(t_r.at[dg.at[0]], drow.at[b], semg3)
                cp1.wait()
                cp2.wait()
                cp3.wait()
                for grp in range(C // 16):
                    rowv = grp * 16 + lax.iota(_I32, 16)
                    svec = sbuf[b, pl.ds(grp * 16, 16)]
                    dvec = dbuf[0, pl.ds(grp * 16, 16)]
                    sl4 = (svec & 31) * 4
                    dl4 = (dvec & 31) * 4
                    sa = plsc.load_gather(ssrow, [z0, rowv, sl4])
                    sb = plsc.load_gather(ssrow, [z0, rowv, sl4 + 1])
                    da = plsc.load_gather(sdrow, [z0, rowv, dl4 + 2])
                    db = plsc.load_gather(sdrow, [z0, rowv, dl4 + 3])
                    ta = sa + da
                    tb = sb + db
                    ea = jnp.exp(-jnp.where(ta >= 0, ta, ALPHA * ta))
                    eb = jnp.exp(-jnp.where(tb >= 0, tb, ALPHA * tb))
                    el = (svec & 7) * 16
                    bv = jnp.full((16,), b, _I32)
                    plsc.store_scatter(erow, [bv, rowv, el], ea)
                    plsc.store_scatter(erow, [bv, rowv, el + 1], eb)
                    for k in range(16):
                        r = grp * 16 + k
                        ka = _bcast_lane(ea, k)
                        kb = _bcast_lane(eb, k)
                        for j in range(4):
                            sl = pl.ds(j * 16, 16)
                            drow[b, r, sl] = drow[b, r, sl] * ka
                        for j in range(4, 8):
                            sl = pl.ds(j * 16, 16)
                            drow[b, r, sl] = drow[b, r, sl] * kb
                pltpu.async_copy(drow.at[b], acct_sh.at[sbuf.at[b]], hsems[b],
                                 add=True)
                pltpu.async_copy(erow.at[b], acct_sh.at[eq.at[b]], esems[b],
                                 add=True)
            return carry

        lax.fori_loop(0, nchunks // 2, pair, 0)
        for b in range(2):
            pltpu.make_async_copy(
                drow.at[b], acct_sh.at[sbuf.at[b]], hsems[b]).wait()
            pltpu.make_async_copy(
                erow.at[b], acct_sh.at[eq.at[b]], esems[b]).wait()
        plsc.subcore_barrier()
        rsl = pl.ds(rbase, rpt)
        pltpu.sync_copy(acct_sh.at[rsl], acct_o.at[cix, rsl])

    return sc_pass


_sc_l1 = _make_sc_pass(core_split=False, use_offset=True, ept=EPT1)
_sc_l2 = _make_sc_pass(core_split=True, use_offset=False, ept=EPT2)


def _extract_rowsums(acct):
    er = acct[:, N_ACC:, :].reshape(2, NE_ROWS, 8, 16)
    return er[..., :2].reshape(2, N_ACC, 2)


def kernel(x, edges, Wl, bl, Wh, ah, Wend, aend):
    src = edges[0]
    dst = edges[1]
    npad = E_PAD - E
    srcp = jnp.concatenate([src, jnp.full((npad,), DUMMY, _I32)])
    dstp = jnp.concatenate([dst, jnp.full((npad,), DUMMY, _I32)])
    xp = jnp.zeros((N_ACC, D_IN), _F32).at[:N].set(x)
    wlt = Wl.T
    blr = bl.reshape(1, D_IN)

    a1 = jnp.zeros((2, D_IN, 4), _F32)
    for c in range(2):
        a1 = a1.at[c, 0:64, 0].set(ah[2 * c, :64])
        a1 = a1.at[c, 64:128, 1].set(ah[2 * c + 1, :64])
        a1 = a1.at[c, 0:64, 2].set(ah[2 * c, 64:])
        a1 = a1.at[c, 64:128, 3].set(ah[2 * c + 1, 64:])
    a2 = jnp.zeros((D_IN, 4), _F32)
    a2 = a2.at[:, 0].set(aend[:128]).at[:, 1].set(aend[:128])
    a2 = a2.at[:, 2].set(aend[128:]).at[:, 3].set(aend[128:])
    bmat = jnp.zeros((2, 128), _F32)
    bmat = bmat.at[0, :64].set(1.0).at[1, 64:].set(1.0)
    b0 = jnp.zeros((2, 128), _F32).at[0, :].set(1.0)

    t1, s1 = _tc_pre(xp, wlt, blr, Wh, a1)
    s1p = s1.reshape(2 * (N_ACC // 32), 128)
    acct1 = _sc_l1(srcp, dstp, t1.reshape(2 * N_ACC, 128), s1p)
    rs1 = _extract_rowsums(acct1)
    t2, s2 = _tc_mid(acct1[:, :N_ACC, :], rs1, Wend.reshape(2, 128, 128),
                     a2, bmat)
    s2p = s2.reshape(2 * (N_ACC // 32), 128)
    acct2 = _sc_l2(srcp, dstp, t2, s2p)
    rs2 = _extract_rowsums(acct2)
    return _tc_post(acct2[:, :N_ACC, :], rs2, b0)
